# baseline (speedup 1.0000x reference)
import functools

import jax
import jax.numpy as jnp
from jax import lax
from jax.experimental import pallas as pl
from jax.experimental.pallas import tpu as pltpu
from jax.experimental.pallas import tpu_sc as plsc

_PI = 3.141592653589793

NUM_ENT = 1000000
NUM_REL = 1000
D = 64
B = 16384
NC = 2
NS = 16
NW = NC * NS
BPW = B // NW
CHUNK = 128
NCHUNK = BPW // CHUNK


def _cos_body(rel_ref, out_ref):
    out_ref[...] = jnp.cos(rel_ref[...] * jnp.float32(_PI))


def _cos_table(rel_table):
    return pl.pallas_call(
        _cos_body,
        out_shape=jax.ShapeDtypeStruct((NUM_REL, D), jnp.float32),
    )(rel_table)


_GATHER_DNUMS = lax.GatherDimensionNumbers(
    offset_dims=(), collapsed_slice_dims=(0,), start_index_map=(0,))


def _rotate16(v, sh):
    idx = (lax.iota(jnp.int32, 16) + sh) & 15
    return lax.gather(v, idx[:, None], _GATHER_DNUMS, (1,),
                      indices_are_sorted=False, unique_indices=False,
                      mode=lax.GatherScatterMode.PROMISE_IN_BOUNDS)


def _hsum16(v):
    for sh in (8, 4, 2, 1):
        v = v + _rotate16(v, sh)
    return v


def _sqrt16(x):
    i = lax.bitcast_convert_type(x, jnp.int32)
    y = lax.bitcast_convert_type(jnp.int32(0x5F3759DF) - (i >> 1), jnp.float32)
    xh = 0.5 * x
    for _ in range(3):
        y = y * (1.5 - xh * y * y)
    return x * y


_MESH = plsc.VectorSubcoreMesh(core_axis_name="c", subcore_axis_name="s")


@functools.partial(
    pl.kernel,
    mesh=_MESH,
    compiler_params=pltpu.CompilerParams(use_tc_tiling_on_sc=False),
    out_type=jax.ShapeDtypeStruct((B,), jnp.float32),
    scratch_types=[
        pltpu.VMEM((NCHUNK, CHUNK), jnp.int32),
        pltpu.VMEM((NCHUNK, CHUNK), jnp.int32),
        pltpu.VMEM((NCHUNK, CHUNK), jnp.int32),
        pltpu.VMEM((BPW, D), jnp.float32),
        pltpu.VMEM((BPW, D), jnp.float32),
        pltpu.VMEM((BPW, D), jnp.float32),
        pltpu.VMEM((BPW,), jnp.float32),
        pltpu.SemaphoreType.DMA,
    ],
)
def _sc_score(ent_hbm, cos_hbm, hidx_hbm, ridx_hbm, tidx_hbm, out_hbm,
              hidx_v, ridx_v, tidx_v, h_rows, c_rows, t_rows, out_v, sem):
    wid = lax.axis_index("s") * NC + lax.axis_index("c")
    base = pl.multiple_of(wid * BPW, BPW)

    pltpu.sync_copy(hidx_hbm.at[wid], hidx_v)
    pltpu.sync_copy(ridx_hbm.at[wid], ridx_v)
    pltpu.sync_copy(tidx_hbm.at[wid], tidx_v)

    copies = []
    for k in range(NCHUNK):
        dst = pl.ds(k * CHUNK, CHUNK)
        copies.append(pltpu.async_copy(ent_hbm.at[hidx_v.at[k]], h_rows.at[dst], sem))
        copies.append(pltpu.async_copy(ent_hbm.at[tidx_v.at[k]], t_rows.at[dst], sem))
        copies.append(pltpu.async_copy(cos_hbm.at[ridx_v.at[k]], c_rows.at[dst], sem))
    for c in copies:
        c.wait()

    lanes = lax.iota(jnp.int32, 16)

    def group(g, carry):
        row0 = pl.multiple_of(g * 16, 16)
        ov = jnp.zeros((16,), jnp.float32)
        for rr in range(16):
            i = row0 + rr
            acc = jnp.zeros((16,), jnp.float32)
            for j in range(D // 16):
                sl = pl.ds(j * 16, 16)
                hv = h_rows[i, sl]
                tv = t_rows[i, sl]
                cv = c_rows[i, sl]
                x = hv * hv + tv * tv - 2.0 * (hv * tv) * cv
                acc = acc + _sqrt16(jnp.maximum(x, 0.0))
            ov = jnp.where(lanes == rr, -_hsum16(acc), ov)
        out_v[pl.ds(row0, 16)] = ov
        return carry

    lax.fori_loop(0, BPW // 16, group, 0)
    pltpu.sync_copy(out_v, out_hbm.at[pl.ds(base, BPW)])


def kernel(h_idx, r_idx, t_idx, ent_table, rel_table):
    cos_table = _cos_table(rel_table)
    h3 = h_idx.astype(jnp.int32).reshape(NW, NCHUNK, CHUNK)
    r3 = r_idx.astype(jnp.int32).reshape(NW, NCHUNK, CHUNK)
    t3 = t_idx.astype(jnp.int32).reshape(NW, NCHUNK, CHUNK)
    return _sc_score(ent_table, cos_table, h3, r3, t3)
